# speedup vs baseline: 8.6055x; 8.6055x over previous
import functools
import math

import jax
import jax.numpy as jnp
from jax import lax
from jax.experimental import pallas as pl
from jax.experimental.pallas import tpu as pltpu
from jax.experimental.pallas import tpu_sc as plsc

VOCAB = 100000
EMBED = 128
MAXLEN = 512
BATCH = 1024
SEQ = 512
SCALE = 2.0 * math.sqrt(EMBED)

N = BATCH * SEQ
LANES = 16
GROUPS = EMBED // LANES


def _make_pe():
    position = jnp.arange(0, MAXLEN, dtype=jnp.float32)[:, None]
    div_term = jnp.exp(
        jnp.arange(0, EMBED, 2, dtype=jnp.float32) * (-math.log(10000.0) / EMBED)
    )
    pe = jnp.zeros((MAXLEN, EMBED), dtype=jnp.float32)
    pe = pe.at[:, 0::2].set(jnp.sin(position * div_term))
    pe = pe.at[:, 1::2].set(jnp.cos(position * div_term))
    return pe


def _build_sc_kernel(nw: int, chunk: int):
    per_w = N // nw
    nch = per_w // chunk
    s_chunks = SEQ // chunk

    mesh = plsc.VectorSubcoreMesh(core_axis_name="c", subcore_axis_name="s")

    @functools.partial(
        pl.kernel,
        mesh=mesh,
        out_type=jax.ShapeDtypeStruct((N, EMBED), jnp.float32),
        scratch_types=[
            pltpu.VMEM((chunk,), jnp.int32),
            pltpu.VMEM((chunk,), jnp.int32),
            pltpu.VMEM((chunk,), jnp.int32),
            pltpu.VMEM((chunk, EMBED), jnp.float32),
            pltpu.VMEM((chunk, EMBED), jnp.float32),
            pltpu.SemaphoreType.DMA,
        ],
    )
    def k(idx_hbm, seg_hbm, tok_hbm, cmb_hbm, out_hbm,
          idx_v, seg_v, cidx_v, tok_v, cmb_v, sem):
        wid = lax.axis_index("s") * 2 + lax.axis_index("c")
        base = wid * per_w
        lane = lax.iota(jnp.int32, LANES)

        def chunk_body(j, carry):
            off = base + j * chunk
            pltpu.sync_copy(idx_hbm.at[pl.ds(off, chunk)], idx_v)
            pltpu.sync_copy(seg_hbm.at[pl.ds(off, chunk)], seg_v)

            s0 = lax.rem(j, s_chunks) * chunk
            for i in range(chunk // LANES):
                s_vec = (s0 + i * LANES) + lane
                g = seg_v[pl.ds(i * LANES, LANES)]
                cidx_v[pl.ds(i * LANES, LANES)] = s_vec * 3 + g

            cp1 = pltpu.async_copy(tok_hbm.at[idx_v], tok_v, sem)
            cp2 = pltpu.async_copy(cmb_hbm.at[cidx_v], cmb_v, sem)
            cp1.wait()
            cp2.wait()

            def row_body(r, carry2):
                for kk in range(GROUPS):
                    t = tok_v[r, pl.ds(kk * LANES, LANES)]
                    c = cmb_v[r, pl.ds(kk * LANES, LANES)]
                    tok_v[r, pl.ds(kk * LANES, LANES)] = SCALE * t + c
                return carry2

            lax.fori_loop(0, chunk, row_body, 0)

            pltpu.sync_copy(tok_v, out_hbm.at[pl.ds(off, chunk)])
            return carry

        lax.fori_loop(0, nch, chunk_body, 0)

    return k


@jax.jit
def kernel(bert_inputs, segment_labels, token_table, segment_table):
    pe = _make_pe()
    comb = (pe[:, None, :] + segment_table[None, :, :]).reshape(3 * MAXLEN, EMBED)

    idx = bert_inputs.reshape(N).astype(jnp.int32)
    seg = segment_labels.reshape(N).astype(jnp.int32)

    k = _build_sc_kernel(nw=32, chunk=128)
    out = k(idx, seg, token_table, comb)
    return out.reshape(BATCH, SEQ, EMBED)
